# Initial kernel scaffold; baseline (speedup 1.0000x reference)
import jax
import jax.numpy as jnp
from jax.experimental import pallas as pl


def kernel(x, edge_index, W, b):
    raise NotImplementedError("write your pallas kernel here")


# speedup vs baseline: 71.1220x; 71.1220x over previous
import functools

import jax
import jax.numpy as jnp
from jax import lax
from jax.experimental import pallas as pl
from jax.experimental.pallas import tpu as pltpu
from jax.experimental.pallas import tpu_sc as plsc

_N = 10000
_E = 320000
_D = 128
_C = 2

_NC, _NS, _L = 2, 16, 16
_NW = _NC * _NS
_NPAD = 10240
_SLICE = _NPAD // _NS
_CHUNK = 128
_NCH = -(-(_E // _NW) // _CHUNK)
_EPT = _NCH * _CHUNK
_EPAD = _EPT * _NW

_MESH = plsc.VectorSubcoreMesh(
    core_axis_name="c", subcore_axis_name="s", num_cores=_NC, num_subcores=_NS
)


def _zero_fill(buf, n):
    def body(i, _):
        buf[pl.ds(i * _L, _L)] = jnp.zeros((_L,), jnp.float32)
        return 0
    lax.fori_loop(0, n // _L, body, 0, unroll=True)


def _hist_body(dst_hbm, cnt_hbm, dst_v, ones_v, zb_v, cnt_sh):
    cid = lax.axis_index("c")
    sid = lax.axis_index("s")
    wid = cid * _NS + sid

    _zero_fill(zb_v, _SLICE)

    def ones_fill(i, _):
        ones_v[pl.ds(i * _L, _L)] = jnp.ones((_L,), jnp.float32)
        return 0
    lax.fori_loop(0, _CHUNK // _L, ones_fill, 0, unroll=True)

    pltpu.sync_copy(zb_v, cnt_sh.at[pl.ds(sid * _SLICE, _SLICE)])
    pltpu.sync_copy(dst_hbm.at[wid], dst_v)
    plsc.subcore_barrier()

    def chunk(j, _):
        pltpu.sync_copy(ones_v, cnt_sh.at[dst_v.at[j]], add=True)
        return 0
    lax.fori_loop(0, _NCH, chunk, 0)

    plsc.subcore_barrier()
    pltpu.sync_copy(
        cnt_sh.at[pl.ds(sid * _SLICE, _SLICE)],
        cnt_hbm.at[pl.ds(cid * _NPAD + sid * _SLICE, _SLICE)],
    )


_SC_PARAMS = pltpu.CompilerParams(needs_layout_passes=False)

_sc_hist = pl.kernel(
    _hist_body,
    out_type=jax.ShapeDtypeStruct((_NC * _NPAD,), jnp.float32),
    mesh=_MESH,
    compiler_params=_SC_PARAMS,
    scratch_types=[
        pltpu.VMEM((_NCH, _CHUNK), jnp.int32),
        pltpu.VMEM((_CHUNK,), jnp.float32),
        pltpu.VMEM((_SLICE,), jnp.float32),
        pltpu.VMEM_SHARED((_NPAD,), jnp.float32),
    ],
)


def _agg_body(src_hbm, dst_hbm, g0_hbm, g1_hbm, p0_hbm, p1_hbm,
              src_v, dst_v, g0_v, g1_v, msg0_v, msg1_v, zb_v,
              acc0_sh, acc1_sh):
    cid = lax.axis_index("c")
    sid = lax.axis_index("s")
    wid = cid * _NS + sid

    _zero_fill(zb_v, _SLICE)
    pltpu.sync_copy(zb_v, acc0_sh.at[pl.ds(sid * _SLICE, _SLICE)])
    pltpu.sync_copy(zb_v, acc1_sh.at[pl.ds(sid * _SLICE, _SLICE)])
    pltpu.sync_copy(g0_hbm, g0_v)
    pltpu.sync_copy(g1_hbm, g1_v)
    pltpu.sync_copy(src_hbm.at[wid], src_v)
    pltpu.sync_copy(dst_hbm.at[wid], dst_v)
    plsc.subcore_barrier()

    def chunk(j, _):
        def gath(k, _):
            sidx = src_v[j, pl.ds(k * _L, _L)]
            msg0_v[pl.ds(k * _L, _L)] = plsc.load_gather(g0_v, [sidx])
            msg1_v[pl.ds(k * _L, _L)] = plsc.load_gather(g1_v, [sidx])
            return 0
        lax.fori_loop(0, _CHUNK // _L, gath, 0, unroll=True)
        pltpu.sync_copy(msg0_v, acc0_sh.at[dst_v.at[j]], add=True)
        pltpu.sync_copy(msg1_v, acc1_sh.at[dst_v.at[j]], add=True)
        return 0
    lax.fori_loop(0, _NCH, chunk, 0)

    plsc.subcore_barrier()
    out_ds = pl.ds(cid * _NPAD + sid * _SLICE, _SLICE)
    sh_ds = pl.ds(sid * _SLICE, _SLICE)
    pltpu.sync_copy(acc0_sh.at[sh_ds], p0_hbm.at[out_ds])
    pltpu.sync_copy(acc1_sh.at[sh_ds], p1_hbm.at[out_ds])


_sc_agg = pl.kernel(
    _agg_body,
    out_type=[
        jax.ShapeDtypeStruct((_NC * _NPAD,), jnp.float32),
        jax.ShapeDtypeStruct((_NC * _NPAD,), jnp.float32),
    ],
    mesh=_MESH,
    compiler_params=_SC_PARAMS,
    scratch_types=[
        pltpu.VMEM((_NCH, _CHUNK), jnp.int32),
        pltpu.VMEM((_NCH, _CHUNK), jnp.int32),
        pltpu.VMEM((_NPAD,), jnp.float32),
        pltpu.VMEM((_NPAD,), jnp.float32),
        pltpu.VMEM((_CHUNK,), jnp.float32),
        pltpu.VMEM((_CHUNK,), jnp.float32),
        pltpu.VMEM((_SLICE,), jnp.float32),
        pltpu.VMEM_SHARED((_NPAD,), jnp.float32),
        pltpu.VMEM_SHARED((_NPAD,), jnp.float32),
    ],
)


def _prep_body(x_ref, w0_ref, w1_ref, cnt_ref, g0_ref, g1_ref, dinv_ref):
    cnt = cnt_ref[0, :] + cnt_ref[1, :]
    dinv = lax.rsqrt(cnt + 1.0)
    x = x_ref[...]
    h0 = jnp.sum(x * w0_ref[...][None, :], axis=1)
    h1 = jnp.sum(x * w1_ref[...][None, :], axis=1)
    dinv_ref[...] = dinv
    g0_ref[...] = h0 * dinv
    g1_ref[...] = h1 * dinv


_tc_prep = pl.pallas_call(
    _prep_body,
    out_shape=[
        jax.ShapeDtypeStruct((_NPAD,), jnp.float32),
        jax.ShapeDtypeStruct((_NPAD,), jnp.float32),
        jax.ShapeDtypeStruct((_NPAD,), jnp.float32),
    ],
)


def _comb_body(p0_ref, p1_ref, g0_ref, g1_ref, dinv_ref, b_ref, out_ref):
    dinv = dinv_ref[...]
    c0 = (p0_ref[0, :] + p0_ref[1, :] + g0_ref[...]) * dinv + b_ref[0]
    c1 = (p1_ref[0, :] + p1_ref[1, :] + g1_ref[...]) * dinv + b_ref[1]
    out = jnp.stack([c0, c1], axis=-1)
    out_ref[...] = out[:_N, :]


_tc_comb = pl.pallas_call(
    _comb_body,
    out_shape=jax.ShapeDtypeStruct((_N, _C), jnp.float32),
    in_specs=[
        pl.BlockSpec(memory_space=pltpu.VMEM),
        pl.BlockSpec(memory_space=pltpu.VMEM),
        pl.BlockSpec(memory_space=pltpu.VMEM),
        pl.BlockSpec(memory_space=pltpu.VMEM),
        pl.BlockSpec(memory_space=pltpu.VMEM),
        pl.BlockSpec(memory_space=pltpu.SMEM),
    ],
)


def kernel(x, edge_index, W, b):
    src = edge_index[0]
    dst = edge_index[1]
    pad = jnp.full((_EPAD - _E,), _N, dtype=jnp.int32)
    src_p = jnp.concatenate([src, pad]).reshape(_NW, _NCH, _CHUNK)
    dst_p = jnp.concatenate([dst, pad]).reshape(_NW, _NCH, _CHUNK)
    x_pad = jnp.pad(x, ((0, _NPAD - _N), (0, 0)))

    cnt = _sc_hist(dst_p).reshape(_NC, _NPAD)
    g0, g1, dinv = _tc_prep(x_pad, W[:, 0], W[:, 1], cnt)
    p0, p1 = _sc_agg(src_p, dst_p, g0, g1)
    return _tc_comb(p0.reshape(_NC, _NPAD), p1.reshape(_NC, _NPAD),
                    g0, g1, dinv, b)


# speedup vs baseline: 100.5213x; 1.4134x over previous
import functools

import jax
import jax.numpy as jnp
from jax import lax
from jax.experimental import pallas as pl
from jax.experimental.pallas import tpu as pltpu
from jax.experimental.pallas import tpu_sc as plsc

_N = 10000
_E = 320000
_D = 128
_C = 2

_NC, _NS, _L = 2, 16, 16
_NW = _NC * _NS
_NPAD = 10240
_SLICE = _NPAD // _NS
_EPT = _E // _NW
_CHUNK = 2000
_NCH = _EPT // _CHUNK

_MESH = plsc.VectorSubcoreMesh(
    core_axis_name="c", subcore_axis_name="s", num_cores=_NC, num_subcores=_NS
)


def _fill(buf, n, value):
    def body(i, _):
        buf[pl.ds(i * _L, _L)] = jnp.full((_L,), value, jnp.float32)
        return 0
    lax.fori_loop(0, n // _L, body, 0)


def _hist_body(edge_hbm, cnt_hbm, dst_v, ones_v, zb_v, cnt_sh):
    cid = lax.axis_index("c")
    sid = lax.axis_index("s")
    wid = cid * _NS + sid

    _fill(zb_v, _SLICE, 0.0)
    _fill(ones_v, _CHUNK, 1.0)

    pltpu.sync_copy(zb_v, cnt_sh.at[pl.ds(sid * _SLICE, _SLICE)])
    pltpu.sync_copy(edge_hbm.at[pl.ds(_E + wid * _EPT, _EPT)], dst_v)
    plsc.subcore_barrier()

    def chunk(j, _):
        pltpu.sync_copy(
            ones_v, cnt_sh.at[dst_v.at[pl.ds(j * _CHUNK, _CHUNK)]], add=True
        )
        return 0
    lax.fori_loop(0, _NCH, chunk, 0)

    plsc.subcore_barrier()
    pltpu.sync_copy(
        cnt_sh.at[pl.ds(sid * _SLICE, _SLICE)],
        cnt_hbm.at[pl.ds(cid * _NPAD + sid * _SLICE, _SLICE)],
    )


_SC_PARAMS = pltpu.CompilerParams(needs_layout_passes=False)

_sc_hist = pl.kernel(
    _hist_body,
    out_type=jax.ShapeDtypeStruct((_NC * _NPAD,), jnp.float32),
    mesh=_MESH,
    compiler_params=_SC_PARAMS,
    scratch_types=[
        pltpu.VMEM((_EPT,), jnp.int32),
        pltpu.VMEM((_CHUNK,), jnp.float32),
        pltpu.VMEM((_SLICE,), jnp.float32),
        pltpu.VMEM_SHARED((_NPAD,), jnp.float32),
    ],
)


def _agg_body(edge_hbm, g0_hbm, g1_hbm, p0_hbm, p1_hbm,
              src_v, dst_v, g0_v, g1_v, msg0_v, msg1_v, zb_v,
              acc0_sh, acc1_sh):
    cid = lax.axis_index("c")
    sid = lax.axis_index("s")
    wid = cid * _NS + sid

    _fill(zb_v, _SLICE, 0.0)
    pltpu.sync_copy(zb_v, acc0_sh.at[pl.ds(sid * _SLICE, _SLICE)])
    pltpu.sync_copy(zb_v, acc1_sh.at[pl.ds(sid * _SLICE, _SLICE)])
    pltpu.sync_copy(g0_hbm, g0_v)
    pltpu.sync_copy(g1_hbm, g1_v)
    pltpu.sync_copy(edge_hbm.at[pl.ds(wid * _EPT, _EPT)], src_v)
    pltpu.sync_copy(edge_hbm.at[pl.ds(_E + wid * _EPT, _EPT)], dst_v)
    plsc.subcore_barrier()

    def chunk(j, _):
        def gath(k, _):
            sidx = src_v[pl.ds(j * _CHUNK + k * _L, _L)]
            msg0_v[pl.ds(k * _L, _L)] = plsc.load_gather(g0_v, [sidx])
            msg1_v[pl.ds(k * _L, _L)] = plsc.load_gather(g1_v, [sidx])
            return 0
        lax.fori_loop(0, _CHUNK // _L, gath, 0)
        didx = dst_v.at[pl.ds(j * _CHUNK, _CHUNK)]
        pltpu.sync_copy(msg0_v, acc0_sh.at[didx], add=True)
        pltpu.sync_copy(msg1_v, acc1_sh.at[didx], add=True)
        return 0
    lax.fori_loop(0, _NCH, chunk, 0)

    plsc.subcore_barrier()
    out_ds = pl.ds(cid * _NPAD + sid * _SLICE, _SLICE)
    sh_ds = pl.ds(sid * _SLICE, _SLICE)
    pltpu.sync_copy(acc0_sh.at[sh_ds], p0_hbm.at[out_ds])
    pltpu.sync_copy(acc1_sh.at[sh_ds], p1_hbm.at[out_ds])


_sc_agg = pl.kernel(
    _agg_body,
    out_type=[
        jax.ShapeDtypeStruct((_NC * _NPAD,), jnp.float32),
        jax.ShapeDtypeStruct((_NC * _NPAD,), jnp.float32),
    ],
    mesh=_MESH,
    compiler_params=_SC_PARAMS,
    scratch_types=[
        pltpu.VMEM((_EPT,), jnp.int32),
        pltpu.VMEM((_EPT,), jnp.int32),
        pltpu.VMEM((_NPAD,), jnp.float32),
        pltpu.VMEM((_NPAD,), jnp.float32),
        pltpu.VMEM((_CHUNK,), jnp.float32),
        pltpu.VMEM((_CHUNK,), jnp.float32),
        pltpu.VMEM((_SLICE,), jnp.float32),
        pltpu.VMEM_SHARED((_NPAD,), jnp.float32),
        pltpu.VMEM_SHARED((_NPAD,), jnp.float32),
    ],
)


def _prep_body(x_ref, w0_ref, w1_ref, cnt_ref, g0_ref, g1_ref, dinv_ref):
    cnt = cnt_ref[0, :] + cnt_ref[1, :]
    dinv = lax.rsqrt(cnt + 1.0)
    x = x_ref[...]
    h0 = jnp.sum(x * w0_ref[...][None, :], axis=1)
    h1 = jnp.sum(x * w1_ref[...][None, :], axis=1)
    dinv_ref[...] = dinv
    g0_ref[...] = jnp.pad(h0, (0, _NPAD - _N)) * dinv
    g1_ref[...] = jnp.pad(h1, (0, _NPAD - _N)) * dinv


_tc_prep = pl.pallas_call(
    _prep_body,
    out_shape=[
        jax.ShapeDtypeStruct((_NPAD,), jnp.float32),
        jax.ShapeDtypeStruct((_NPAD,), jnp.float32),
        jax.ShapeDtypeStruct((_NPAD,), jnp.float32),
    ],
)


def _comb_body(p0_ref, p1_ref, g0_ref, g1_ref, dinv_ref, b_ref, out_ref):
    dinv = dinv_ref[...]
    c0 = (p0_ref[0, :] + p0_ref[1, :] + g0_ref[...]) * dinv + b_ref[0]
    c1 = (p1_ref[0, :] + p1_ref[1, :] + g1_ref[...]) * dinv + b_ref[1]
    out = jnp.stack([c0, c1], axis=-1)
    out_ref[...] = out[:_N, :]


_tc_comb = pl.pallas_call(
    _comb_body,
    out_shape=jax.ShapeDtypeStruct((_N, _C), jnp.float32),
    in_specs=[
        pl.BlockSpec(memory_space=pltpu.VMEM),
        pl.BlockSpec(memory_space=pltpu.VMEM),
        pl.BlockSpec(memory_space=pltpu.VMEM),
        pl.BlockSpec(memory_space=pltpu.VMEM),
        pl.BlockSpec(memory_space=pltpu.VMEM),
        pl.BlockSpec(memory_space=pltpu.SMEM),
    ],
)


def kernel(x, edge_index, W, b):
    edges = edge_index.reshape(2 * _E)
    cnt = _sc_hist(edges).reshape(_NC, _NPAD)
    g0, g1, dinv = _tc_prep(x, W[:, 0], W[:, 1], cnt)
    p0, p1 = _sc_agg(edges, g0, g1)
    return _tc_comb(p0.reshape(_NC, _NPAD), p1.reshape(_NC, _NPAD),
                    g0, g1, dinv, b)


# speedup vs baseline: 122.2647x; 1.2163x over previous
import functools

import jax
import jax.numpy as jnp
from jax import lax
from jax.experimental import pallas as pl
from jax.experimental.pallas import tpu as pltpu
from jax.experimental.pallas import tpu_sc as plsc

_N = 10000
_E = 320000
_D = 128
_C = 2

_NC, _NS, _L = 2, 16, 16
_NW = _NC * _NS
_NPAD = 10240
_SLICE = _NPAD // _NS
_EPT = _E // _NW
_CHUNK = 2000
_NCH = _EPT // _CHUNK

_MESH = plsc.VectorSubcoreMesh(
    core_axis_name="c", subcore_axis_name="s", num_cores=_NC, num_subcores=_NS
)


def _fill(buf, n, value):
    def body(i, _):
        buf[pl.ds(i * _L, _L)] = jnp.full((_L,), value, jnp.float32)
        return 0
    lax.fori_loop(0, n // _L, body, 0)


def _hist_body(edge_hbm, cnt_hbm, dst_v, ones_v, zb_v, cnt_sh):
    cid = lax.axis_index("c")
    sid = lax.axis_index("s")
    wid = cid * _NS + sid

    _fill(zb_v, _SLICE, 0.0)
    _fill(ones_v, _CHUNK, 1.0)

    pltpu.sync_copy(zb_v, cnt_sh.at[pl.ds(sid * _SLICE, _SLICE)])
    pltpu.sync_copy(edge_hbm.at[pl.ds(_E + wid * _EPT, _EPT)], dst_v)
    plsc.subcore_barrier()

    def chunk(j, _):
        pltpu.sync_copy(
            ones_v,
            cnt_sh.at[dst_v.at[pl.ds(j * _CHUNK, _CHUNK)]],
            add=True,
        )
        return 0
    lax.fori_loop(0, _NCH, chunk, 0)

    plsc.subcore_barrier()
    pltpu.sync_copy(
        cnt_sh.at[pl.ds(sid * _SLICE, _SLICE)],
        cnt_hbm.at[pl.ds(cid * _NPAD + sid * _SLICE, _SLICE)],
    )


_SC_PARAMS = pltpu.CompilerParams(needs_layout_passes=False)

_sc_hist = pl.kernel(
    _hist_body,
    out_type=jax.ShapeDtypeStruct((_NC * _NPAD,), jnp.float32),
    mesh=_MESH,
    compiler_params=_SC_PARAMS,
    scratch_types=[
        pltpu.VMEM((_EPT,), jnp.int32),
        pltpu.VMEM((_CHUNK,), jnp.float32),
        pltpu.VMEM((_SLICE,), jnp.float32),
        pltpu.VMEM_SHARED((_NPAD,), jnp.float32),
    ],
)


def _agg_body(edge_hbm, g0_hbm, g1_hbm, p0_hbm, p1_hbm,
              src_v, dst_v, g0_v, g1_v, msg0_v, msg1_v, zb_v,
              acc0_sh, acc1_sh):
    cid = lax.axis_index("c")
    sid = lax.axis_index("s")
    wid = cid * _NS + sid

    _fill(zb_v, _SLICE, 0.0)
    pltpu.sync_copy(
        [edge_hbm.at[pl.ds(wid * _EPT, _EPT)],
         edge_hbm.at[pl.ds(_E + wid * _EPT, _EPT)],
         g0_hbm, g1_hbm],
        [src_v, dst_v, g0_v, g1_v],
    )
    pltpu.sync_copy(zb_v, acc0_sh.at[pl.ds(sid * _SLICE, _SLICE)])
    pltpu.sync_copy(zb_v, acc1_sh.at[pl.ds(sid * _SLICE, _SLICE)])
    plsc.subcore_barrier()

    def chunk(j, _):
        def grp(k, _):
            sidx = src_v[pl.ds(j * _CHUNK + k * _L, _L)]
            m0 = plsc.load_gather(g0_v, [sidx])
            m1 = plsc.load_gather(g1_v, [sidx])
            msg0_v[pl.ds(k * _L, _L)] = m0
            msg1_v[pl.ds(k * _L, _L)] = m1
            return 0
        lax.fori_loop(0, _CHUNK // _L, grp, 0)
        didx = dst_v.at[pl.ds(j * _CHUNK, _CHUNK)]
        pltpu.sync_copy(
            [msg0_v, msg1_v],
            [acc0_sh.at[didx], acc1_sh.at[didx]],
            add=True,
        )
        return 0
    lax.fori_loop(0, _NCH, chunk, 0)

    plsc.subcore_barrier()
    pltpu.sync_copy(
        [acc0_sh.at[pl.ds(sid * _SLICE, _SLICE)],
         acc1_sh.at[pl.ds(sid * _SLICE, _SLICE)]],
        [p0_hbm.at[pl.ds(cid * _NPAD + sid * _SLICE, _SLICE)],
         p1_hbm.at[pl.ds(cid * _NPAD + sid * _SLICE, _SLICE)]],
    )


_sc_agg = pl.kernel(
    _agg_body,
    out_type=[
        jax.ShapeDtypeStruct((_NC * _NPAD,), jnp.float32),
        jax.ShapeDtypeStruct((_NC * _NPAD,), jnp.float32),
    ],
    mesh=_MESH,
    compiler_params=_SC_PARAMS,
    scratch_types=[
        pltpu.VMEM((_EPT,), jnp.int32),
        pltpu.VMEM((_EPT,), jnp.int32),
        pltpu.VMEM((_NPAD,), jnp.float32),
        pltpu.VMEM((_NPAD,), jnp.float32),
        pltpu.VMEM((_CHUNK,), jnp.float32),
        pltpu.VMEM((_CHUNK,), jnp.float32),
        pltpu.VMEM((_SLICE,), jnp.float32),
        pltpu.VMEM_SHARED((_NPAD,), jnp.float32),
        pltpu.VMEM_SHARED((_NPAD,), jnp.float32),
    ],
)


def _scale_body(h0_ref, h1_ref, cnt_ref, g0_ref, g1_ref, dinv_ref):
    dinv = lax.rsqrt(
        cnt_ref[pl.ds(0, _NPAD)] + cnt_ref[pl.ds(_NPAD, _NPAD)] + 1.0
    )
    dinv_ref[...] = dinv
    g0_ref[...] = h0_ref[...] * dinv
    g1_ref[...] = h1_ref[...] * dinv


_tc_scale = pl.pallas_call(
    _scale_body,
    out_shape=[
        jax.ShapeDtypeStruct((_NPAD,), jnp.float32),
        jax.ShapeDtypeStruct((_NPAD,), jnp.float32),
        jax.ShapeDtypeStruct((_NPAD,), jnp.float32),
    ],
)


def _mm_body(x_ref, w0_ref, w1_ref, h0_ref, h1_ref):
    x = x_ref[...]
    h0 = jnp.sum(x * w0_ref[...][None, :], axis=1)
    h1 = jnp.sum(x * w1_ref[...][None, :], axis=1)
    h0_ref[...] = jnp.pad(h0, (0, _NPAD - _N))
    h1_ref[...] = jnp.pad(h1, (0, _NPAD - _N))


_tc_matmul = pl.pallas_call(
    _mm_body,
    out_shape=[
        jax.ShapeDtypeStruct((_NPAD,), jnp.float32),
        jax.ShapeDtypeStruct((_NPAD,), jnp.float32),
    ],
)


def _comb_body(p0_ref, p1_ref, g0_ref, g1_ref, dinv_ref, b_ref, out_ref):
    dinv = dinv_ref[...]
    s0 = g0_ref[...]
    s1 = g1_ref[...]
    for w in range(_NC):
        s0 = s0 + p0_ref[pl.ds(w * _NPAD, _NPAD)]
        s1 = s1 + p1_ref[pl.ds(w * _NPAD, _NPAD)]
    c0 = s0 * dinv + b_ref[0]
    c1 = s1 * dinv + b_ref[1]
    out = jnp.stack([c0, c1], axis=-1)
    out_ref[...] = out[:_N, :]


_tc_comb = pl.pallas_call(
    _comb_body,
    out_shape=jax.ShapeDtypeStruct((_N, _C), jnp.float32),
    in_specs=[
        pl.BlockSpec(memory_space=pltpu.VMEM),
        pl.BlockSpec(memory_space=pltpu.VMEM),
        pl.BlockSpec(memory_space=pltpu.VMEM),
        pl.BlockSpec(memory_space=pltpu.VMEM),
        pl.BlockSpec(memory_space=pltpu.VMEM),
        pl.BlockSpec(memory_space=pltpu.SMEM),
    ],
)


def kernel(x, edge_index, W, b):
    edge_flat = edge_index.reshape(2 * _E)
    h0, h1 = _tc_matmul(x, W[:, 0], W[:, 1])
    cnt = _sc_hist(edge_flat)
    g0, g1, dinv = _tc_scale(h0, h1, cnt)
    p0, p1 = _sc_agg(edge_flat, g0, g1)
    return _tc_comb(p0, p1, g0, g1, dinv, b)


# speedup vs baseline: 125.2906x; 1.0247x over previous
import functools

import jax
import jax.numpy as jnp
from jax import lax
from jax.experimental import pallas as pl
from jax.experimental.pallas import tpu as pltpu
from jax.experimental.pallas import tpu_sc as plsc

_N = 10000
_E = 320000
_D = 128
_C = 2

_NC, _NS, _L = 2, 16, 16
_NW = _NC * _NS
_NPAD = 10240
_SLICE = _NPAD // _NS
_EPT = _E // _NW
_CHUNK = 2000
_NCH = _EPT // _CHUNK

_MESH = plsc.VectorSubcoreMesh(
    core_axis_name="c", subcore_axis_name="s", num_cores=_NC, num_subcores=_NS
)


def _fill(buf, n, value):
    def body(i, _):
        buf[pl.ds(i * _L, _L)] = jnp.full((_L,), value, jnp.float32)
        return 0
    lax.fori_loop(0, n // _L, body, 0)


def _hist_body(edge_hbm, cnt_hbm, dst_v, ones_v, zb_v, cnt_sh):
    cid = lax.axis_index("c")
    sid = lax.axis_index("s")
    wid = cid * _NS + sid

    _fill(zb_v, _SLICE, 0.0)
    _fill(ones_v, _CHUNK, 1.0)

    pltpu.sync_copy(zb_v, cnt_sh.at[pl.ds(sid * _SLICE, _SLICE)])
    pltpu.sync_copy(edge_hbm.at[pl.ds(_E + wid * _EPT, _EPT)], dst_v)
    plsc.subcore_barrier()

    def scoped(*sems):
        descs = []
        for j in range(_NCH):
            d = pltpu.make_async_copy(
                ones_v,
                cnt_sh.at[dst_v.at[pl.ds(j * _CHUNK, _CHUNK)]],
                sems[j],
            )
            d.start(add=True)
            descs.append(d)
        for d in descs:
            d.wait()

    pl.run_scoped(scoped, *([pltpu.SemaphoreType.DMA(())] * _NCH))

    plsc.subcore_barrier()
    pltpu.sync_copy(
        cnt_sh.at[pl.ds(sid * _SLICE, _SLICE)],
        cnt_hbm.at[pl.ds(cid * _NPAD + sid * _SLICE, _SLICE)],
    )


_SC_PARAMS = pltpu.CompilerParams(needs_layout_passes=False)

_sc_hist = pl.kernel(
    _hist_body,
    out_type=jax.ShapeDtypeStruct((_NC * _NPAD,), jnp.float32),
    mesh=_MESH,
    compiler_params=_SC_PARAMS,
    scratch_types=[
        pltpu.VMEM((_EPT,), jnp.int32),
        pltpu.VMEM((_CHUNK,), jnp.float32),
        pltpu.VMEM((_SLICE,), jnp.float32),
        pltpu.VMEM_SHARED((_NPAD,), jnp.float32),
    ],
)


def _agg_body(edge_hbm, g0_hbm, g1_hbm, p0_hbm, p1_hbm,
              src_v, dst_v, g0_v, g1_v,
              msg0a_v, msg1a_v, msg0b_v, msg1b_v, zb_v,
              acc0_sh, acc1_sh):
    cid = lax.axis_index("c")
    sid = lax.axis_index("s")
    wid = cid * _NS + sid

    _fill(zb_v, _SLICE, 0.0)
    pltpu.sync_copy(
        [edge_hbm.at[pl.ds(wid * _EPT, _EPT)],
         edge_hbm.at[pl.ds(_E + wid * _EPT, _EPT)],
         g0_hbm, g1_hbm],
        [src_v, dst_v, g0_v, g1_v],
    )
    pltpu.sync_copy(zb_v, acc0_sh.at[pl.ds(sid * _SLICE, _SLICE)])
    pltpu.sync_copy(zb_v, acc1_sh.at[pl.ds(sid * _SLICE, _SLICE)])
    plsc.subcore_barrier()

    def gather_chunk(j, m0, m1):
        def grp(k, _):
            for u in range(5):
                off = k * 5 * _L + u * _L
                sidx = src_v[pl.ds(j * _CHUNK + off, _L)]
                m0[pl.ds(off, _L)] = plsc.load_gather(g0_v, [sidx])
                m1[pl.ds(off, _L)] = plsc.load_gather(g1_v, [sidx])
            return 0
        lax.fori_loop(0, _CHUNK // (5 * _L), grp, 0)

    def scoped(*sems):
        bufs = [(msg0a_v, msg1a_v), (msg0b_v, msg1b_v)]
        pending = [None, None]
        for j in range(_NCH):
            p = j % 2
            m0, m1 = bufs[p]
            if pending[p] is not None:
                pending[p][0].wait()
                pending[p][1].wait()
            gather_chunk(j, m0, m1)
            didx = dst_v.at[pl.ds(j * _CHUNK, _CHUNK)]
            d0 = pltpu.make_async_copy(m0, acc0_sh.at[didx], sems[2 * p])
            d1 = pltpu.make_async_copy(m1, acc1_sh.at[didx], sems[2 * p + 1])
            d0.start(add=True)
            d1.start(add=True)
            pending[p] = (d0, d1)
        for p in range(2):
            if pending[p] is not None:
                pending[p][0].wait()
                pending[p][1].wait()

    pl.run_scoped(scoped, *([pltpu.SemaphoreType.DMA(())] * 4))

    plsc.subcore_barrier()
    pltpu.sync_copy(
        [acc0_sh.at[pl.ds(sid * _SLICE, _SLICE)],
         acc1_sh.at[pl.ds(sid * _SLICE, _SLICE)]],
        [p0_hbm.at[pl.ds(cid * _NPAD + sid * _SLICE, _SLICE)],
         p1_hbm.at[pl.ds(cid * _NPAD + sid * _SLICE, _SLICE)]],
    )


_sc_agg = pl.kernel(
    _agg_body,
    out_type=[
        jax.ShapeDtypeStruct((_NC * _NPAD,), jnp.float32),
        jax.ShapeDtypeStruct((_NC * _NPAD,), jnp.float32),
    ],
    mesh=_MESH,
    compiler_params=_SC_PARAMS,
    scratch_types=[
        pltpu.VMEM((_EPT,), jnp.int32),
        pltpu.VMEM((_EPT,), jnp.int32),
        pltpu.VMEM((_NPAD,), jnp.float32),
        pltpu.VMEM((_NPAD,), jnp.float32),
        pltpu.VMEM((_CHUNK,), jnp.float32),
        pltpu.VMEM((_CHUNK,), jnp.float32),
        pltpu.VMEM((_CHUNK,), jnp.float32),
        pltpu.VMEM((_CHUNK,), jnp.float32),
        pltpu.VMEM((_SLICE,), jnp.float32),
        pltpu.VMEM_SHARED((_NPAD,), jnp.float32),
        pltpu.VMEM_SHARED((_NPAD,), jnp.float32),
    ],
)


def _scale_body(h0_ref, h1_ref, cnt_ref, g0_ref, g1_ref, dinv_ref):
    dinv = lax.rsqrt(
        cnt_ref[pl.ds(0, _NPAD)] + cnt_ref[pl.ds(_NPAD, _NPAD)] + 1.0
    )
    dinv_ref[...] = dinv
    g0_ref[...] = h0_ref[...] * dinv
    g1_ref[...] = h1_ref[...] * dinv


_tc_scale = pl.pallas_call(
    _scale_body,
    out_shape=[
        jax.ShapeDtypeStruct((_NPAD,), jnp.float32),
        jax.ShapeDtypeStruct((_NPAD,), jnp.float32),
        jax.ShapeDtypeStruct((_NPAD,), jnp.float32),
    ],
)


def _mm_body(x_ref, w0_ref, w1_ref, h0_ref, h1_ref):
    x = x_ref[...]
    h0 = jnp.sum(x * w0_ref[...][None, :], axis=1)
    h1 = jnp.sum(x * w1_ref[...][None, :], axis=1)
    h0_ref[...] = jnp.pad(h0, (0, _NPAD - _N))
    h1_ref[...] = jnp.pad(h1, (0, _NPAD - _N))


_tc_matmul = pl.pallas_call(
    _mm_body,
    out_shape=[
        jax.ShapeDtypeStruct((_NPAD,), jnp.float32),
        jax.ShapeDtypeStruct((_NPAD,), jnp.float32),
    ],
)


def _comb_body(p0_ref, p1_ref, g0_ref, g1_ref, dinv_ref, b_ref, out_ref):
    dinv = dinv_ref[...]
    s0 = g0_ref[...]
    s1 = g1_ref[...]
    for w in range(_NC):
        s0 = s0 + p0_ref[pl.ds(w * _NPAD, _NPAD)]
        s1 = s1 + p1_ref[pl.ds(w * _NPAD, _NPAD)]
    c0 = s0 * dinv + b_ref[0]
    c1 = s1 * dinv + b_ref[1]
    out = jnp.stack([c0, c1], axis=-1)
    out_ref[...] = out[:_N, :]


_tc_comb = pl.pallas_call(
    _comb_body,
    out_shape=jax.ShapeDtypeStruct((_N, _C), jnp.float32),
    in_specs=[
        pl.BlockSpec(memory_space=pltpu.VMEM),
        pl.BlockSpec(memory_space=pltpu.VMEM),
        pl.BlockSpec(memory_space=pltpu.VMEM),
        pl.BlockSpec(memory_space=pltpu.VMEM),
        pl.BlockSpec(memory_space=pltpu.VMEM),
        pl.BlockSpec(memory_space=pltpu.SMEM),
    ],
)


def kernel(x, edge_index, W, b):
    edge_flat = edge_index.reshape(2 * _E)
    h0, h1 = _tc_matmul(x, W[:, 0], W[:, 1])
    cnt = _sc_hist(edge_flat)
    g0, g1, dinv = _tc_scale(h0, h1, cnt)
    p0, p1 = _sc_agg(edge_flat, g0, g1)
    return _tc_comb(p0, p1, g0, g1, dinv, b)
